# baseline (speedup 1.0000x reference)
import functools

import jax
import jax.numpy as jnp
from jax import lax
from jax.experimental import pallas as pl
from jax.experimental.pallas import tpu as pltpu
from jax.experimental.pallas import tpu_sc as plsc

BATCH = 4096
SEQ = 200
D_EMBED = 32
NUM_WORKERS = 32
BATCH_PER_W = BATCH // NUM_WORKERS
HALF = SEQ // 2
NCHUNKS = BATCH_PER_W * 2
K = 8
NROUNDS = NCHUNKS // K


def _sc_gather(tokens2d, emb):
    mesh = plsc.VectorSubcoreMesh(core_axis_name="c", subcore_axis_name="s")

    @functools.partial(
        pl.kernel,
        mesh=mesh,
        out_type=jax.ShapeDtypeStruct((BATCH, SEQ, D_EMBED), jnp.float32),
        scratch_types=[
            pltpu.VMEM((NCHUNKS, HALF), jnp.int32),
            pltpu.VMEM((K, HALF, D_EMBED), jnp.float32),
            pltpu.SemaphoreType.DMA((K,)),
            pltpu.SemaphoreType.DMA((K,)),
        ],
        compiler_params=pltpu.CompilerParams(use_tc_tiling_on_sc=False),
    )
    def k(tok_hbm, emb_hbm, out_hbm, idx_v, rows_v, gsem, ssem):
        wid = lax.axis_index("s") * 2 + lax.axis_index("c")
        pltpu.sync_copy(tok_hbm.at[pl.ds(wid * NCHUNKS, NCHUNKS)], idx_v)
        batch_base = wid * BATCH_PER_W

        def round_body(g, carry):
            base = g * K
            gathers = []
            for b in range(K):
                gathers.append(pltpu.async_copy(
                    emb_hbm.at[idx_v.at[base + b]], rows_v.at[b], gsem.at[b]))
            stores = []
            for b in range(K):
                c = base + b
                gathers[b].wait()
                stores.append(pltpu.async_copy(
                    rows_v.at[b],
                    out_hbm.at[batch_base + lax.div(c, 2),
                               pl.ds(lax.rem(c, 2) * HALF, HALF)],
                    ssem.at[b]))
            for b in range(K):
                stores[b].wait()
            return carry

        lax.fori_loop(0, NROUNDS, round_body, 0)

    return k(tokens2d, emb)


def kernel(tokens, emb):
    tok2d = tokens.reshape(BATCH * 2, HALF).astype(jnp.int32)
    return _sc_gather(tok2d, emb)
